# baseline (speedup 1.0000x reference)
import functools
import jax
import jax.numpy as jnp
from jax import lax
from jax.experimental import pallas as pl
from jax.experimental.pallas import tpu as pltpu
from jax.experimental.pallas import tpu_sc as plsc


def kernel(x_ids, table):
    bsz, seq_len = x_ids.shape
    d = table.shape[1]

    info = plsc.get_sparse_core_info()
    NC, NS = info.num_cores, info.num_subcores
    NW = NC * NS
    rows_per_w = seq_len // NW
    NBUF = 2
    AHEAD = 1
    buf_rows = (64, 56)
    sizes = []
    left = rows_per_w
    while left > 0:
        c = min(buf_rows[len(sizes) % NBUF], left)
        sizes.append(c)
        left -= c
    assert all(c % 8 == 0 for c in sizes)
    offs = [sum(sizes[:i]) for i in range(len(sizes))]
    n_chunks = len(sizes)

    mesh = plsc.VectorSubcoreMesh(core_axis_name="c", subcore_axis_name="s")

    @functools.partial(
        pl.kernel,
        mesh=mesh,
        out_type=jax.ShapeDtypeStruct((bsz, seq_len, d), table.dtype),
        scratch_types=(
            [pltpu.VMEM_SHARED((NS, r, d), table.dtype) for r in buf_rows]
            + [pltpu.SemaphoreType.DMA]
            + [pltpu.SemaphoreType.DMA for _ in range(NBUF)]
        ),
    )
    def sc_copy(table_hbm, out_hbm, *refs):
        sid = lax.axis_index("s")
        bufs = [refs[n].at[sid] for n in range(NBUF)]
        gsem = refs[NBUF]
        wsems = refs[NBUF + 1 :]
        wid = sid * NC + lax.axis_index("c")
        base = wid * rows_per_w
        gathers = [None] * NBUF
        scatters = [[] for _ in range(NBUF)]

        def gather(j):
            return pltpu.async_copy(
                table_hbm.at[pl.ds(base + offs[j], sizes[j])],
                bufs[j % NBUF].at[pl.ds(0, sizes[j])],
                gsem,
            )

        for j in range(min(AHEAD, n_chunks)):
            gathers[j % NBUF] = gather(j)
        for i in range(n_chunks):
            k = i % NBUF
            gathers[k].wait()
            nxt = i + AHEAD
            if nxt < n_chunks:
                nk = nxt % NBUF
                for h in scatters[nk]:
                    h.wait()
                scatters[nk] = []
                gathers[nk] = gather(nxt)
            start = base + offs[i]
            for b in range(bsz):
                scatters[k].append(
                    pltpu.async_copy(
                        bufs[k].at[pl.ds(0, sizes[i])],
                        out_hbm.at[b, pl.ds(start, sizes[i])],
                        wsems[k],
                    )
                )
        for k in range(NBUF):
            for h in scatters[k]:
                h.wait()

    return sc_copy(table)


# speedup vs baseline: 1.2095x; 1.2095x over previous
import functools
import jax
import jax.numpy as jnp
from jax import lax
from jax.experimental import pallas as pl
from jax.experimental.pallas import tpu as pltpu
from jax.experimental.pallas import tpu_sc as plsc


def kernel(x_ids, table):
    bsz, seq_len = x_ids.shape
    d = table.shape[1]

    info = plsc.get_sparse_core_info()
    NC, NS = info.num_cores, info.num_subcores
    NW = NC * NS
    rows_per_w = seq_len // NW
    NBUF = 2
    AHEAD = 1
    buf_rows = (64, 56)
    sizes = []
    left = rows_per_w
    while left > 0:
        c = min(buf_rows[len(sizes) % NBUF], left)
        sizes.append(c)
        left -= c
    assert all(c % 8 == 0 for c in sizes)
    offs = [sum(sizes[:i]) for i in range(len(sizes))]
    n_chunks = len(sizes)

    mesh = plsc.VectorSubcoreMesh(core_axis_name="c", subcore_axis_name="s")

    @functools.partial(
        pl.kernel,
        mesh=mesh,
        out_type=jax.ShapeDtypeStruct((bsz, seq_len, d), table.dtype),
        scratch_types=(
            [pltpu.VMEM((r, d), table.dtype) for r in buf_rows]
            + [pltpu.SemaphoreType.DMA]
            + [pltpu.SemaphoreType.DMA for _ in range(NBUF)]
        ),
    )
    def sc_copy(table_hbm, out_hbm, *refs):
        bufs = refs[:NBUF]
        gsem = refs[NBUF]
        wsems = refs[NBUF + 1 :]
        wid = lax.axis_index("s") * NC + lax.axis_index("c")
        base = wid * rows_per_w
        gathers = [None] * NBUF
        scatters = [[] for _ in range(NBUF)]

        def gather(j):
            return pltpu.async_copy(
                table_hbm.at[pl.ds(base + offs[j], sizes[j])],
                bufs[j % NBUF].at[pl.ds(0, sizes[j])],
                gsem,
            )

        for j in range(min(AHEAD, n_chunks)):
            gathers[j % NBUF] = gather(j)
        for i in range(n_chunks):
            k = i % NBUF
            gathers[k].wait()
            nxt = i + AHEAD
            if nxt < n_chunks:
                nk = nxt % NBUF
                for h in scatters[nk]:
                    h.wait()
                scatters[nk] = []
                gathers[nk] = gather(nxt)
            start = base + offs[i]
            for b in range(bsz):
                scatters[k].append(
                    pltpu.async_copy(
                        bufs[k].at[pl.ds(0, sizes[i])],
                        out_hbm.at[b, pl.ds(start, sizes[i])],
                        wsems[k],
                    )
                )
        for k in range(NBUF):
            for h in scatters[k]:
                h.wait()

    return sc_copy(table)
